# baseline (speedup 1.0000x reference)
import functools

import jax
import jax.numpy as jnp
from jax import lax
from jax.experimental import pallas as pl
from jax.experimental.pallas import tpu as pltpu
from jax.experimental.pallas import tpu_sc as plsc

B_, S_, D_IN, D_OUT, E_, K_ = 2, 2048, 1024, 1024, 8, 2
T_ = B_ * S_
A_ = T_ * K_
M_ = 256
NPAD = A_ + E_ * M_
NBLK = NPAD // M_

NC, NS, L = 2, 16, 16
APC = A_ // NS
SPW = NPAD // (NC * NS)
GCH = 40
TPW = T_ // (NC * NS)


def _gating_body(x_ref, gw_ref, gb_ref, s_ref, i_ref):
  x = x_ref[...]
  gw = gw_ref[...]
  logits = lax.dot_general(x, gw, (((1,), (1,)), ((), ())),
                           preferred_element_type=jnp.float32)
  logits = logits + gb_ref[...]
  m = jnp.max(logits, axis=1, keepdims=True)
  p = jnp.exp(logits - m)
  denom = jnp.sum(p, axis=1)
  iota = lax.broadcasted_iota(jnp.int32, p.shape, 1)
  v0 = jnp.max(p, axis=1)
  i0 = jnp.min(jnp.where(p == v0[:, None], iota, E_), axis=1)
  pm = jnp.where(iota == i0[:, None], -1.0, p)
  v1 = jnp.max(pm, axis=1)
  i1 = jnp.min(jnp.where(pm == v1[:, None], iota, E_), axis=1)
  s_ref[0, :] = v0 / denom
  s_ref[1, :] = v1 / denom
  i_ref[0, :] = i0
  i_ref[1, :] = i1


def _gating(x2d, gate_W, gate_b2d):
  tb = 512
  grid = (T_ // tb,)
  return pl.pallas_call(
      _gating_body,
      grid=grid,
      in_specs=[
          pl.BlockSpec((tb, D_IN), lambda i: (i, 0)),
          pl.BlockSpec((E_, D_IN), lambda i: (0, 0)),
          pl.BlockSpec((1, E_), lambda i: (0, 0)),
      ],
      out_specs=[
          pl.BlockSpec((K_, tb), lambda i: (0, i)),
          pl.BlockSpec((K_, tb), lambda i: (0, i)),
      ],
      out_shape=[
          jax.ShapeDtypeStruct((K_, T_), jnp.float32),
          jax.ShapeDtypeStruct((K_, T_), jnp.int32),
      ],
  )(x2d, gate_W, gate_b2d)


def _vfull(val, dtype=jnp.int32):
  return jnp.full((L,), val, dtype)


def _route_body(keys_hbm, scores_hbm, x_hbm,
                xs_hbm, ss_hbm, pos_hbm, blk_hbm, vld_hbm,
                keys_v, scores_v, pos2_v, tok2_v, sc2_v,
                h_v, hist_l, z_v,
                idxa_v, idxb_v, rowsa_v, rowsb_v, blk_v, vld_v,
                hist_sh, tok_sh, s_sh, sem, semb, semsa, semsb):
  cid = lax.axis_index("c")
  sid = lax.axis_index("s")
  lanes = lax.iota(jnp.int32, L)

  pltpu.sync_copy(keys_hbm.at[pl.ds(sid * APC, APC)], keys_v)
  for j in range(APC // 128):
    pltpu.sync_copy(scores_hbm.at[pl.ds(sid * APC + j * 128, 128)],
                    sc2_v.at[j])

  def hist_step(r, h):
    kv = keys_v[pl.ds(r * L, L)]
    for e in range(E_):
      cnt = jnp.sum(jnp.where(kv == e, 1, 0))
      h = h + jnp.where(lanes == e, cnt, 0)
    return h
  h = lax.fori_loop(0, APC // L, hist_step, jnp.zeros((L,), jnp.int32))
  h_v[...] = h
  pltpu.sync_copy(h_v, hist_sh.at[pl.ds(sid * L, L)])

  def z_step(r, _):
    z_v[pl.ds(r * L, L)] = jnp.zeros((L,), jnp.int32)
    return 0
  lax.fori_loop(0, (NPAD // NS) // L, z_step, 0)
  pltpu.sync_copy(z_v, tok_sh.at[pl.ds(sid * (NPAD // NS), NPAD // NS)])

  plsc.subcore_barrier()

  pltpu.sync_copy(hist_sh, hist_l)
  total = jnp.zeros((L,), jnp.int32)
  prior = jnp.zeros((L,), jnp.int32)
  sid_v = jnp.full((L,), sid, jnp.int32)
  for w in range(NS):
    v = hist_l[pl.ds(w * L, L)]
    total = total + v
    prior = prior + jnp.where(_vfull(w) < sid_v, v, 0)
  padded = jnp.bitwise_and(total + (M_ - 1), -M_)
  cs = plsc.cumsum(padded)
  base = cs - padded
  start = base + prior

  start_s = [jnp.sum(jnp.where(lanes == e, start, 0)) for e in range(E_)]
  base_s = [jnp.sum(jnp.where(lanes == e, base, 0)) for e in range(E_)]
  padded_s = [jnp.sum(jnp.where(lanes == e, padded, 0)) for e in range(E_)]

  def place_step(r, running):
    kv = keys_v[pl.ds(r * L, L)]
    jv = sid * APC + r * L + lanes
    posv = jnp.zeros((L,), jnp.int32)
    new_running = []
    for e in range(E_):
      mask = kv == e
      mi = jnp.where(mask, 1, 0)
      pre = plsc.cumsum(mi) - mi
      slot = start_s[e] + running[e] + pre
      posv = jnp.where(mask, slot, posv)
      new_running.append(running[e] + jnp.sum(mi))
    row = r // 8
    col = (r % 8) * L
    tok2_v[row, pl.ds(col, L)] = jnp.bitwise_and(jv, T_ - 1)
    pos2_v[row, pl.ds(col, L)] = posv
    return tuple(new_running)
  lax.fori_loop(0, APC // L, place_step,
                tuple(jnp.zeros((), jnp.int32) for _ in range(E_)))

  for j in range(APC // 128):
    pltpu.sync_copy(tok2_v.at[j], tok_sh.at[pos2_v.at[j]])
    pltpu.sync_copy(sc2_v.at[j], s_sh.at[pos2_v.at[j]])

  @pl.when(cid == 0)
  def _():
    for j in range(APC // 128):
      pltpu.sync_copy(pos2_v.at[j], pos_hbm.at[pl.ds(sid * APC + j * 128, 128)])

  @pl.when((cid + sid) == 0)
  def _():
    padend = jnp.sum(jnp.where(lanes == E_ - 1, cs, 0))
    for g in range(NBLK // L + 1):
      blkstart = (g * L + lanes) * M_
      acc = jnp.zeros((L,), jnp.int32)
      for e in range(E_):
        ge = jnp.where(blkstart >= base_s[e], 1, 0)
        lt = jnp.where(blkstart < base_s[e] + padded_s[e], 1, 0)
        acc = acc + e * ge * lt
      blk_v[pl.ds(g * L, L)] = acc
      vld_v[pl.ds(g * L, L)] = jnp.where(blkstart < padend, 1, 0)
    pltpu.sync_copy(blk_v, blk_hbm)
    pltpu.sync_copy(vld_v, vld_hbm)

  plsc.subcore_barrier()

  off = (cid * NS + sid) * SPW
  pltpu.sync_copy(s_sh.at[pl.ds(off, SPW)], scores_v.at[pl.ds(0, SPW)])
  pltpu.sync_copy(scores_v.at[pl.ds(0, SPW)], ss_hbm.at[pl.ds(off, SPW)])

  pltpu.sync_copy(tok_sh.at[pl.ds(off, GCH)], idxa_v)
  pltpu.async_copy(x_hbm.at[idxa_v], rowsa_v, sem)

  def gather_step(i, _):
    c0 = off + (2 * i) * GCH
    c1 = off + (2 * i + 1) * GCH
    pltpu.sync_copy(tok_sh.at[pl.ds(c1, GCH)], idxb_v)
    pltpu.async_copy(x_hbm.at[idxb_v], rowsb_v, semb)
    pltpu.make_async_copy(x_hbm.at[idxa_v], rowsa_v, sem).wait()
    pltpu.async_copy(rowsa_v, xs_hbm.at[pl.ds(c0, GCH)], semsa)
    pltpu.make_async_copy(x_hbm.at[idxb_v], rowsb_v, semb).wait()
    pltpu.async_copy(rowsb_v, xs_hbm.at[pl.ds(c1, GCH)], semsb)

    @pl.when(i < SPW // GCH // 2 - 1)
    def _():
      pltpu.sync_copy(tok_sh.at[pl.ds(c1 + GCH, GCH)], idxa_v)
    pltpu.make_async_copy(rowsa_v, xs_hbm.at[pl.ds(c0, GCH)], semsa).wait()
    pltpu.make_async_copy(rowsb_v, xs_hbm.at[pl.ds(c1, GCH)], semsb).wait()

    @pl.when(i < SPW // GCH // 2 - 1)
    def _():
      pltpu.async_copy(x_hbm.at[idxa_v], rowsa_v, sem)
    return 0
  lax.fori_loop(0, SPW // GCH // 2, gather_step, 0)


def _route_gather(keys, scores, x2d):
  mesh = plsc.VectorSubcoreMesh(core_axis_name="c", subcore_axis_name="s",
                                num_cores=NC, num_subcores=NS)
  f = pl.kernel(
      _route_body,
      out_type=[
          jax.ShapeDtypeStruct((NPAD, D_IN), jnp.float32),
          jax.ShapeDtypeStruct((NPAD,), jnp.float32),
          jax.ShapeDtypeStruct((A_,), jnp.int32),
          jax.ShapeDtypeStruct((NBLK + L,), jnp.int32),
          jax.ShapeDtypeStruct((NBLK + L,), jnp.int32),
      ],
      mesh=mesh,
      scratch_types=[
          pltpu.VMEM((APC,), jnp.int32),
          pltpu.VMEM((APC,), jnp.float32),
          pltpu.VMEM((APC // 128, 128), jnp.int32),
          pltpu.VMEM((APC // 128, 128), jnp.int32),
          pltpu.VMEM((APC // 128, 128), jnp.float32),
          pltpu.VMEM((L,), jnp.int32),
          pltpu.VMEM((NS * L,), jnp.int32),
          pltpu.VMEM((NPAD // NS,), jnp.int32),
          pltpu.VMEM((GCH,), jnp.int32),
          pltpu.VMEM((GCH,), jnp.int32),
          pltpu.VMEM((GCH, D_IN), jnp.float32),
          pltpu.VMEM((GCH, D_IN), jnp.float32),
          pltpu.VMEM((NBLK + L,), jnp.int32),
          pltpu.VMEM((NBLK + L,), jnp.int32),
          pltpu.VMEM_SHARED((NS * L,), jnp.int32),
          pltpu.VMEM_SHARED((NPAD,), jnp.int32),
          pltpu.VMEM_SHARED((NPAD,), jnp.float32),
          pltpu.SemaphoreType.DMA,
          pltpu.SemaphoreType.DMA,
          pltpu.SemaphoreType.DMA,
          pltpu.SemaphoreType.DMA,
      ],
      compiler_params=pltpu.CompilerParams(needs_layout_passes=False),
  )
  return f(keys, scores, x2d)


def _gmm_body(be_ref, vl_ref, x_ref, w_ref, b_ref, s_ref, o_ref):
  @pl.when(vl_ref[pl.program_id(0)] == 1)
  def _():
    xb = x_ref[...]
    wb = w_ref[...][0]
    acc = lax.dot_general(xb, wb, (((1,), (1,)), ((), ())),
                          preferred_element_type=jnp.float32)
    o_ref[...] = (acc + b_ref[...][0]) * s_ref[...][0, 0][:, None]


def _gmm(blk_e, vld, xs, expert_W, expert_b, ss):
  grid_spec = pltpu.PrefetchScalarGridSpec(
      num_scalar_prefetch=2,
      grid=(NBLK,),
      in_specs=[
          pl.BlockSpec((M_, D_IN), lambda i, be, vl: (i, 0)),
          pl.BlockSpec((1, D_OUT, D_IN), lambda i, be, vl: (be[i], 0, 0)),
          pl.BlockSpec((1, 1, D_OUT), lambda i, be, vl: (be[i], 0, 0)),
          pl.BlockSpec((1, 1, M_), lambda i, be, vl: (i, 0, 0)),
      ],
      out_specs=pl.BlockSpec((M_, D_OUT), lambda i, be, vl: (i, 0)),
  )
  return pl.pallas_call(
      _gmm_body,
      grid_spec=grid_spec,
      out_shape=jax.ShapeDtypeStruct((NPAD, D_OUT), jnp.float32),
  )(blk_e, vld, xs, expert_W, expert_b.reshape(E_, 1, D_OUT),
    ss.reshape(NBLK, 1, M_))


def _combine_body(y_hbm, pos_hbm, out_hbm,
                  i0_v, i1_v, r0_v, r1_v, ob_v, sem0, sem1):
  cid = lax.axis_index("c")
  sid = lax.axis_index("s")
  tb = (cid * NS + sid) * TPW

  def step(c, _):
    t0 = tb + c * L
    pltpu.sync_copy(pos_hbm.at[pl.ds(t0, L)], i0_v)
    pltpu.sync_copy(pos_hbm.at[pl.ds(T_ + t0, L)], i1_v)
    cp0 = pltpu.async_copy(y_hbm.at[i0_v], r0_v, sem0)
    cp1 = pltpu.async_copy(y_hbm.at[i1_v], r1_v, sem1)
    cp0.wait()
    cp1.wait()

    def add_col(cc, _):
      for rr in range(L):
        ob_v[rr, pl.ds(cc * L, L)] = (r0_v[rr, pl.ds(cc * L, L)] +
                                      r1_v[rr, pl.ds(cc * L, L)])
      return 0
    lax.fori_loop(0, D_OUT // L, add_col, 0)
    pltpu.sync_copy(ob_v, out_hbm.at[pl.ds(t0, L)])
    return 0
  lax.fori_loop(0, TPW // L, step, 0)


def _combine(y, pos):
  mesh = plsc.VectorSubcoreMesh(core_axis_name="c", subcore_axis_name="s",
                                num_cores=NC, num_subcores=NS)
  f = pl.kernel(
      _combine_body,
      out_type=jax.ShapeDtypeStruct((T_, D_OUT), jnp.float32),
      mesh=mesh,
      scratch_types=[
          pltpu.VMEM((L,), jnp.int32),
          pltpu.VMEM((L,), jnp.int32),
          pltpu.VMEM((L, D_OUT), jnp.float32),
          pltpu.VMEM((L, D_OUT), jnp.float32),
          pltpu.VMEM((L, D_OUT), jnp.float32),
          pltpu.SemaphoreType.DMA,
          pltpu.SemaphoreType.DMA,
      ],
      compiler_params=pltpu.CompilerParams(needs_layout_passes=False),
  )
  return f(y, pos)


@jax.jit
def kernel(x, expert_W, expert_b, gate_W, gate_b):
  x2d = x.reshape(T_, D_IN)
  scores, idx = _gating(x2d, gate_W, gate_b.reshape(1, E_))
  keys = idx.reshape(A_)
  scores_flat = scores.reshape(A_)
  xs, ss, pos, blk, vld = _route_gather(keys, scores_flat, x2d)
  y = _gmm(blk[:NBLK], vld[:NBLK], xs, expert_W, expert_b, ss)
  out = _combine(y, pos)
  return out.reshape(B_, S_, D_OUT)


# speedup vs baseline: 1.0067x; 1.0067x over previous
import functools

import jax
import jax.numpy as jnp
from jax import lax
from jax.experimental import pallas as pl
from jax.experimental.pallas import tpu as pltpu
from jax.experimental.pallas import tpu_sc as plsc

B_, S_, D_IN, D_OUT, E_, K_ = 2, 2048, 1024, 1024, 8, 2
T_ = B_ * S_
A_ = T_ * K_
M_ = 256
NPAD = A_ + E_ * M_
NBLK = NPAD // M_

NC, NS, L = 2, 16, 16
APC = A_ // NS
SPW = NPAD // (NC * NS)
GCH = 40
TPW = T_ // (NC * NS)


def _gating_body(x_ref, gw_ref, gb_ref, s_ref, i_ref):
  x = x_ref[...]
  gw = gw_ref[...]
  logits = lax.dot_general(x, gw, (((1,), (1,)), ((), ())),
                           preferred_element_type=jnp.float32)
  logits = logits + gb_ref[...]
  m = jnp.max(logits, axis=1, keepdims=True)
  p = jnp.exp(logits - m)
  denom = jnp.sum(p, axis=1)
  iota = lax.broadcasted_iota(jnp.int32, p.shape, 1)
  v0 = jnp.max(p, axis=1)
  i0 = jnp.min(jnp.where(p == v0[:, None], iota, E_), axis=1)
  pm = jnp.where(iota == i0[:, None], -1.0, p)
  v1 = jnp.max(pm, axis=1)
  i1 = jnp.min(jnp.where(pm == v1[:, None], iota, E_), axis=1)
  s_ref[0, :] = v0 / denom
  s_ref[1, :] = v1 / denom
  i_ref[0, :] = i0
  i_ref[1, :] = i1


def _gating(x2d, gate_W, gate_b2d):
  tb = 512
  grid = (T_ // tb,)
  return pl.pallas_call(
      _gating_body,
      grid=grid,
      in_specs=[
          pl.BlockSpec((tb, D_IN), lambda i: (i, 0)),
          pl.BlockSpec((E_, D_IN), lambda i: (0, 0)),
          pl.BlockSpec((1, E_), lambda i: (0, 0)),
      ],
      out_specs=[
          pl.BlockSpec((K_, tb), lambda i: (0, i)),
          pl.BlockSpec((K_, tb), lambda i: (0, i)),
      ],
      out_shape=[
          jax.ShapeDtypeStruct((K_, T_), jnp.float32),
          jax.ShapeDtypeStruct((K_, T_), jnp.int32),
      ],
  )(x2d, gate_W, gate_b2d)


def _vfull(val, dtype=jnp.int32):
  return jnp.full((L,), val, dtype)


def _route_body(keys_hbm, scores_hbm, x_hbm,
                xs_hbm, ss_hbm, pos_hbm, blk_hbm, vld_hbm,
                keys_v, scores_v, pos2_v, tok2_v, sc2_v,
                h_v, hist_l, z_v,
                tokloc_v, rowsa_v, rowsb_v, blk_v, vld_v,
                hist_sh, tok_sh, s_sh, sem, semb, semsa, semsb):
  cid = lax.axis_index("c")
  sid = lax.axis_index("s")
  lanes = lax.iota(jnp.int32, L)

  pltpu.sync_copy(keys_hbm.at[pl.ds(sid * APC, APC)], keys_v)
  for j in range(APC // 128):
    pltpu.sync_copy(scores_hbm.at[pl.ds(sid * APC + j * 128, 128)],
                    sc2_v.at[j])

  def hist_step(r, h):
    kv = keys_v[pl.ds(r * L, L)]
    for e in range(E_):
      cnt = jnp.sum(jnp.where(kv == e, 1, 0))
      h = h + jnp.where(lanes == e, cnt, 0)
    return h
  h = lax.fori_loop(0, APC // L, hist_step, jnp.zeros((L,), jnp.int32))
  h_v[...] = h
  pltpu.sync_copy(h_v, hist_sh.at[pl.ds(sid * L, L)])

  def z_step(r, _):
    z_v[pl.ds(r * L, L)] = jnp.zeros((L,), jnp.int32)
    return 0
  lax.fori_loop(0, (NPAD // NS) // L, z_step, 0)
  pltpu.sync_copy(z_v, tok_sh.at[pl.ds(sid * (NPAD // NS), NPAD // NS)])

  plsc.subcore_barrier()

  pltpu.sync_copy(hist_sh, hist_l)
  total = jnp.zeros((L,), jnp.int32)
  prior = jnp.zeros((L,), jnp.int32)
  sid_v = jnp.full((L,), sid, jnp.int32)
  for w in range(NS):
    v = hist_l[pl.ds(w * L, L)]
    total = total + v
    prior = prior + jnp.where(_vfull(w) < sid_v, v, 0)
  padded = jnp.bitwise_and(total + (M_ - 1), -M_)
  cs = plsc.cumsum(padded)
  base = cs - padded
  start = base + prior

  start_s = [jnp.sum(jnp.where(lanes == e, start, 0)) for e in range(E_)]
  base_s = [jnp.sum(jnp.where(lanes == e, base, 0)) for e in range(E_)]
  padded_s = [jnp.sum(jnp.where(lanes == e, padded, 0)) for e in range(E_)]

  def place_step(r, running):
    kv = keys_v[pl.ds(r * L, L)]
    jv = sid * APC + r * L + lanes
    posv = jnp.zeros((L,), jnp.int32)
    new_running = []
    for e in range(E_):
      mask = kv == e
      mi = jnp.where(mask, 1, 0)
      pre = plsc.cumsum(mi) - mi
      slot = start_s[e] + running[e] + pre
      posv = jnp.where(mask, slot, posv)
      new_running.append(running[e] + jnp.sum(mi))
    row = r // 8
    col = (r % 8) * L
    tok2_v[row, pl.ds(col, L)] = jnp.bitwise_and(jv, T_ - 1)
    pos2_v[row, pl.ds(col, L)] = posv
    return tuple(new_running)
  lax.fori_loop(0, APC // L, place_step,
                tuple(jnp.zeros((), jnp.int32) for _ in range(E_)))

  for j in range(APC // 128):
    pltpu.sync_copy(tok2_v.at[j], tok_sh.at[pos2_v.at[j]])
    pltpu.sync_copy(sc2_v.at[j], s_sh.at[pos2_v.at[j]])

  @pl.when(cid == 0)
  def _():
    for j in range(APC // 128):
      pltpu.sync_copy(pos2_v.at[j], pos_hbm.at[pl.ds(sid * APC + j * 128, 128)])

  @pl.when((cid + sid) == 0)
  def _():
    padend = jnp.sum(jnp.where(lanes == E_ - 1, cs, 0))
    for g in range(NBLK // L + 1):
      blkstart = (g * L + lanes) * M_
      acc = jnp.zeros((L,), jnp.int32)
      for e in range(E_):
        ge = jnp.where(blkstart >= base_s[e], 1, 0)
        lt = jnp.where(blkstart < base_s[e] + padded_s[e], 1, 0)
        acc = acc + e * ge * lt
      blk_v[pl.ds(g * L, L)] = acc
      vld_v[pl.ds(g * L, L)] = jnp.where(blkstart < padend, 1, 0)
    pltpu.sync_copy(blk_v, blk_hbm)
    pltpu.sync_copy(vld_v, vld_hbm)

  plsc.subcore_barrier()

  off = (cid * NS + sid) * SPW
  pltpu.sync_copy(s_sh.at[pl.ds(off, SPW)], scores_v.at[pl.ds(0, SPW)])
  pltpu.sync_copy(scores_v.at[pl.ds(0, SPW)], ss_hbm.at[pl.ds(off, SPW)])

  pltpu.sync_copy(tok_sh.at[pl.ds(off, SPW)], tokloc_v)
  pltpu.async_copy(x_hbm.at[tokloc_v.at[pl.ds(0, GCH)]], rowsa_v, sem)

  def gather_step(i, _):
    c0 = (2 * i) * GCH
    c1 = (2 * i + 1) * GCH
    pltpu.async_copy(x_hbm.at[tokloc_v.at[pl.ds(c1, GCH)]], rowsb_v, semb)
    pltpu.make_async_copy(x_hbm.at[tokloc_v.at[pl.ds(c0, GCH)]],
                          rowsa_v, sem).wait()
    pltpu.async_copy(rowsa_v, xs_hbm.at[pl.ds(off + c0, GCH)], semsa)
    pltpu.make_async_copy(x_hbm.at[tokloc_v.at[pl.ds(c1, GCH)]],
                          rowsb_v, semb).wait()
    pltpu.async_copy(rowsb_v, xs_hbm.at[pl.ds(off + c1, GCH)], semsb)
    pltpu.make_async_copy(rowsa_v, xs_hbm.at[pl.ds(off + c0, GCH)],
                          semsa).wait()
    pltpu.make_async_copy(rowsb_v, xs_hbm.at[pl.ds(off + c1, GCH)],
                          semsb).wait()

    @pl.when(i < SPW // GCH // 2 - 1)
    def _():
      pltpu.async_copy(x_hbm.at[tokloc_v.at[pl.ds(c1 + GCH, GCH)]],
                       rowsa_v, sem)
    return 0
  lax.fori_loop(0, SPW // GCH // 2, gather_step, 0)


def _route_gather(keys, scores, x2d):
  mesh = plsc.VectorSubcoreMesh(core_axis_name="c", subcore_axis_name="s",
                                num_cores=NC, num_subcores=NS)
  f = pl.kernel(
      _route_body,
      out_type=[
          jax.ShapeDtypeStruct((NPAD, D_IN), jnp.float32),
          jax.ShapeDtypeStruct((NPAD,), jnp.float32),
          jax.ShapeDtypeStruct((A_,), jnp.int32),
          jax.ShapeDtypeStruct((NBLK + L,), jnp.int32),
          jax.ShapeDtypeStruct((NBLK + L,), jnp.int32),
      ],
      mesh=mesh,
      scratch_types=[
          pltpu.VMEM((APC,), jnp.int32),
          pltpu.VMEM((APC,), jnp.float32),
          pltpu.VMEM((APC // 128, 128), jnp.int32),
          pltpu.VMEM((APC // 128, 128), jnp.int32),
          pltpu.VMEM((APC // 128, 128), jnp.float32),
          pltpu.VMEM((L,), jnp.int32),
          pltpu.VMEM((NS * L,), jnp.int32),
          pltpu.VMEM((NPAD // NS,), jnp.int32),
          pltpu.VMEM((SPW,), jnp.int32),
          pltpu.VMEM((GCH, D_IN), jnp.float32),
          pltpu.VMEM((GCH, D_IN), jnp.float32),
          pltpu.VMEM((NBLK + L,), jnp.int32),
          pltpu.VMEM((NBLK + L,), jnp.int32),
          pltpu.VMEM_SHARED((NS * L,), jnp.int32),
          pltpu.VMEM_SHARED((NPAD,), jnp.int32),
          pltpu.VMEM_SHARED((NPAD,), jnp.float32),
          pltpu.SemaphoreType.DMA,
          pltpu.SemaphoreType.DMA,
          pltpu.SemaphoreType.DMA,
          pltpu.SemaphoreType.DMA,
      ],
      compiler_params=pltpu.CompilerParams(needs_layout_passes=False),
  )
  return f(keys, scores, x2d)


def _gmm_body(be_ref, vl_ref, x_ref, w_ref, b_ref, s_ref, o_ref):
  @pl.when(vl_ref[pl.program_id(0)] == 1)
  def _():
    xb = x_ref[...]
    wb = w_ref[...][0]
    acc = lax.dot_general(xb, wb, (((1,), (1,)), ((), ())),
                          preferred_element_type=jnp.float32)
    o_ref[...] = (acc + b_ref[...][0]) * s_ref[...][0, 0][:, None]


def _gmm(blk_e, vld, xs, expert_W, expert_b, ss):
  grid_spec = pltpu.PrefetchScalarGridSpec(
      num_scalar_prefetch=2,
      grid=(NBLK,),
      in_specs=[
          pl.BlockSpec((M_, D_IN), lambda i, be, vl: (i, 0)),
          pl.BlockSpec((1, D_OUT, D_IN), lambda i, be, vl: (be[i], 0, 0)),
          pl.BlockSpec((1, 1, D_OUT), lambda i, be, vl: (be[i], 0, 0)),
          pl.BlockSpec((1, 1, M_), lambda i, be, vl: (i, 0, 0)),
      ],
      out_specs=pl.BlockSpec((M_, D_OUT), lambda i, be, vl: (i, 0)),
  )
  return pl.pallas_call(
      _gmm_body,
      grid_spec=grid_spec,
      out_shape=jax.ShapeDtypeStruct((NPAD, D_OUT), jnp.float32),
  )(blk_e, vld, xs, expert_W, expert_b.reshape(E_, 1, D_OUT),
    ss.reshape(NBLK, 1, M_))


def _combine_body(y_hbm, pos_hbm, out_hbm,
                  i0_v, i1_v, r0_v, r1_v, ob_v, sem0, sem1):
  cid = lax.axis_index("c")
  sid = lax.axis_index("s")
  tb = (cid * NS + sid) * TPW

  def step(c, _):
    t0 = tb + c * L
    pltpu.sync_copy(pos_hbm.at[pl.ds(t0, L)], i0_v)
    pltpu.sync_copy(pos_hbm.at[pl.ds(T_ + t0, L)], i1_v)
    cp0 = pltpu.async_copy(y_hbm.at[i0_v], r0_v, sem0)
    cp1 = pltpu.async_copy(y_hbm.at[i1_v], r1_v, sem1)
    cp0.wait()
    cp1.wait()

    def add_col(cc, _):
      for rr in range(L):
        ob_v[rr, pl.ds(cc * L, L)] = (r0_v[rr, pl.ds(cc * L, L)] +
                                      r1_v[rr, pl.ds(cc * L, L)])
      return 0
    lax.fori_loop(0, D_OUT // L, add_col, 0)
    pltpu.sync_copy(ob_v, out_hbm.at[pl.ds(t0, L)])
    return 0
  lax.fori_loop(0, TPW // L, step, 0)


def _combine(y, pos):
  mesh = plsc.VectorSubcoreMesh(core_axis_name="c", subcore_axis_name="s",
                                num_cores=NC, num_subcores=NS)
  f = pl.kernel(
      _combine_body,
      out_type=jax.ShapeDtypeStruct((T_, D_OUT), jnp.float32),
      mesh=mesh,
      scratch_types=[
          pltpu.VMEM((L,), jnp.int32),
          pltpu.VMEM((L,), jnp.int32),
          pltpu.VMEM((L, D_OUT), jnp.float32),
          pltpu.VMEM((L, D_OUT), jnp.float32),
          pltpu.VMEM((L, D_OUT), jnp.float32),
          pltpu.SemaphoreType.DMA,
          pltpu.SemaphoreType.DMA,
      ],
      compiler_params=pltpu.CompilerParams(needs_layout_passes=False),
  )
  return f(y, pos)


@jax.jit
def kernel(x, expert_W, expert_b, gate_W, gate_b):
  x2d = x.reshape(T_, D_IN)
  scores, idx = _gating(x2d, gate_W, gate_b.reshape(1, E_))
  keys = idx.reshape(A_)
  scores_flat = scores.reshape(A_)
  xs, ss, pos, blk, vld = _route_gather(keys, scores_flat, x2d)
  y = _gmm(blk[:NBLK], vld[:NBLK], xs, expert_W, expert_b, ss)
  out = _combine(y, pos)
  return out.reshape(B_, S_, D_OUT)


# speedup vs baseline: 1.2690x; 1.2606x over previous
import functools

import jax
import jax.numpy as jnp
from jax import lax
from jax.experimental import pallas as pl
from jax.experimental.pallas import tpu as pltpu
from jax.experimental.pallas import tpu_sc as plsc

B_, S_, D_IN, D_OUT, E_, K_ = 2, 2048, 1024, 1024, 8, 2
T_ = B_ * S_
A_ = T_ * K_
M_ = 256
NPAD = A_ + E_ * M_
NBLK = NPAD // M_

NC, NS, L = 2, 16, 16
APC = A_ // NS
SPW = NPAD // (NC * NS)
GCH = 40
TPW = T_ // (NC * NS)


def _gating_body(x_ref, gw_ref, gb_ref, s_ref, i_ref):
  x = x_ref[...]
  gw = gw_ref[...]
  logits = lax.dot_general(x, gw, (((1,), (1,)), ((), ())),
                           preferred_element_type=jnp.float32)
  logits = logits + gb_ref[...]
  m = jnp.max(logits, axis=1, keepdims=True)
  p = jnp.exp(logits - m)
  denom = jnp.sum(p, axis=1)
  iota = lax.broadcasted_iota(jnp.int32, p.shape, 1)
  v0 = jnp.max(p, axis=1)
  i0 = jnp.min(jnp.where(p == v0[:, None], iota, E_), axis=1)
  pm = jnp.where(iota == i0[:, None], -1.0, p)
  v1 = jnp.max(pm, axis=1)
  i1 = jnp.min(jnp.where(pm == v1[:, None], iota, E_), axis=1)
  s_ref[0, :] = v0 / denom
  s_ref[1, :] = v1 / denom
  i_ref[0, :] = i0
  i_ref[1, :] = i1


def _gating(x2d, gate_W, gate_b2d):
  tb = 512
  grid = (T_ // tb,)
  return pl.pallas_call(
      _gating_body,
      grid=grid,
      in_specs=[
          pl.BlockSpec((tb, D_IN), lambda i: (i, 0)),
          pl.BlockSpec((E_, D_IN), lambda i: (0, 0)),
          pl.BlockSpec((1, E_), lambda i: (0, 0)),
      ],
      out_specs=[
          pl.BlockSpec((K_, tb), lambda i: (0, i)),
          pl.BlockSpec((K_, tb), lambda i: (0, i)),
      ],
      out_shape=[
          jax.ShapeDtypeStruct((K_, T_), jnp.float32),
          jax.ShapeDtypeStruct((K_, T_), jnp.int32),
      ],
  )(x2d, gate_W, gate_b2d)


def _vfull(val, dtype=jnp.int32):
  return jnp.full((L,), val, dtype)


def _route_body(keys_hbm, scores_hbm, x_hbm,
                xs_hbm, ss_hbm, pos_hbm, blk_hbm, vld_hbm,
                keys_v, scores_v, pos2_v, tok2_v, sc2_v,
                h_v, hist_l, z_v,
                tokloc_v, rowsa_v, rowsb_v, blk_v, vld_v,
                hist_sh, tok_sh, s_sh, sem, semb, semsa, semsb):
  cid = lax.axis_index("c")
  sid = lax.axis_index("s")
  lanes = lax.iota(jnp.int32, L)

  pltpu.sync_copy(keys_hbm.at[pl.ds(sid * APC, APC)], keys_v)
  for j in range(APC // 128):
    pltpu.sync_copy(scores_hbm.at[pl.ds(sid * APC + j * 128, 128)],
                    sc2_v.at[j])

  def hist_step(r, h):
    kv = keys_v[pl.ds(r * L, L)]
    for e in range(E_):
      cnt = jnp.sum(jnp.where(kv == e, 1, 0))
      h = h + jnp.where(lanes == e, cnt, 0)
    return h
  h = lax.fori_loop(0, APC // L, hist_step, jnp.zeros((L,), jnp.int32))
  h_v[...] = h
  pltpu.sync_copy(h_v, hist_sh.at[pl.ds(sid * L, L)])

  def z_step(r, _):
    z_v[pl.ds(r * L, L)] = jnp.zeros((L,), jnp.int32)
    return 0
  lax.fori_loop(0, (NPAD // NS) // L, z_step, 0)
  pltpu.sync_copy(z_v, tok_sh.at[pl.ds(sid * (NPAD // NS), NPAD // NS)])

  plsc.subcore_barrier()

  pltpu.sync_copy(hist_sh, hist_l)
  total = jnp.zeros((L,), jnp.int32)
  prior = jnp.zeros((L,), jnp.int32)
  sid_v = jnp.full((L,), sid, jnp.int32)
  for w in range(NS):
    v = hist_l[pl.ds(w * L, L)]
    total = total + v
    prior = prior + jnp.where(_vfull(w) < sid_v, v, 0)
  padded = jnp.bitwise_and(total + (M_ - 1), -M_)
  cs = plsc.cumsum(padded)
  base = cs - padded
  start = base + prior

  start_s = [jnp.sum(jnp.where(lanes == e, start, 0)) for e in range(E_)]
  base_s = [jnp.sum(jnp.where(lanes == e, base, 0)) for e in range(E_)]
  padded_s = [jnp.sum(jnp.where(lanes == e, padded, 0)) for e in range(E_)]

  def place_step(r, running):
    kv = keys_v[pl.ds(r * L, L)]
    jv = sid * APC + r * L + lanes
    posv = jnp.zeros((L,), jnp.int32)
    new_running = []
    for e in range(E_):
      mask = kv == e
      mi = jnp.where(mask, 1, 0)
      pre = plsc.cumsum(mi) - mi
      slot = start_s[e] + running[e] + pre
      posv = jnp.where(mask, slot, posv)
      new_running.append(running[e] + jnp.sum(mi))
    row = r // 8
    col = (r % 8) * L
    tok2_v[row, pl.ds(col, L)] = jnp.bitwise_and(jv, T_ - 1)
    pos2_v[row, pl.ds(col, L)] = posv
    return tuple(new_running)
  lax.fori_loop(0, APC // L, place_step,
                tuple(jnp.zeros((), jnp.int32) for _ in range(E_)))

  for j in range(APC // 128):
    pltpu.sync_copy(tok2_v.at[j], tok_sh.at[pos2_v.at[j]])
    pltpu.sync_copy(sc2_v.at[j], s_sh.at[pos2_v.at[j]])

  @pl.when(cid == 0)
  def _():
    for j in range(APC // 128):
      pltpu.sync_copy(pos2_v.at[j], pos_hbm.at[pl.ds(sid * APC + j * 128, 128)])

  @pl.when((cid + sid) == 0)
  def _():
    padend = jnp.sum(jnp.where(lanes == E_ - 1, cs, 0))
    for g in range(NBLK // L + 1):
      blkstart = (g * L + lanes) * M_
      acc = jnp.zeros((L,), jnp.int32)
      for e in range(E_):
        ge = jnp.where(blkstart >= base_s[e], 1, 0)
        lt = jnp.where(blkstart < base_s[e] + padded_s[e], 1, 0)
        acc = acc + e * ge * lt
      blk_v[pl.ds(g * L, L)] = acc
      vld_v[pl.ds(g * L, L)] = jnp.where(blkstart < padend, 1, 0)
    pltpu.sync_copy(blk_v, blk_hbm)
    pltpu.sync_copy(vld_v, vld_hbm)

  plsc.subcore_barrier()

  off = (cid * NS + sid) * SPW
  pltpu.sync_copy(s_sh.at[pl.ds(off, SPW)], scores_v.at[pl.ds(0, SPW)])
  pltpu.sync_copy(scores_v.at[pl.ds(0, SPW)], ss_hbm.at[pl.ds(off, SPW)])

  padend = jnp.sum(jnp.where(lanes == E_ - 1, cs, 0))
  wid = cid * NS + sid

  def gather_step(j, _):
    sl = (wid + NC * NS * j) * GCH

    @pl.when(sl < padend)
    def _():
      pltpu.sync_copy(tok_sh.at[pl.ds(sl, GCH)], tokloc_v)
      pltpu.async_copy(x_hbm.at[tokloc_v], rowsa_v, sem).wait()
      pltpu.sync_copy(rowsa_v, xs_hbm.at[pl.ds(sl, GCH)])
    return 0
  lax.fori_loop(0, SPW // GCH, gather_step, 0)


def _route_gather(keys, scores, x2d):
  mesh = plsc.VectorSubcoreMesh(core_axis_name="c", subcore_axis_name="s",
                                num_cores=NC, num_subcores=NS)
  f = pl.kernel(
      _route_body,
      out_type=[
          jax.ShapeDtypeStruct((NPAD, D_IN), jnp.float32),
          jax.ShapeDtypeStruct((NPAD,), jnp.float32),
          jax.ShapeDtypeStruct((A_,), jnp.int32),
          jax.ShapeDtypeStruct((NBLK + L,), jnp.int32),
          jax.ShapeDtypeStruct((NBLK + L,), jnp.int32),
      ],
      mesh=mesh,
      scratch_types=[
          pltpu.VMEM((APC,), jnp.int32),
          pltpu.VMEM((APC,), jnp.float32),
          pltpu.VMEM((APC // 128, 128), jnp.int32),
          pltpu.VMEM((APC // 128, 128), jnp.int32),
          pltpu.VMEM((APC // 128, 128), jnp.float32),
          pltpu.VMEM((L,), jnp.int32),
          pltpu.VMEM((NS * L,), jnp.int32),
          pltpu.VMEM((NPAD // NS,), jnp.int32),
          pltpu.VMEM((GCH,), jnp.int32),
          pltpu.VMEM((GCH, D_IN), jnp.float32),
          pltpu.VMEM((GCH, D_IN), jnp.float32),
          pltpu.VMEM((NBLK + L,), jnp.int32),
          pltpu.VMEM((NBLK + L,), jnp.int32),
          pltpu.VMEM_SHARED((NS * L,), jnp.int32),
          pltpu.VMEM_SHARED((NPAD,), jnp.int32),
          pltpu.VMEM_SHARED((NPAD,), jnp.float32),
          pltpu.SemaphoreType.DMA,
          pltpu.SemaphoreType.DMA,
          pltpu.SemaphoreType.DMA,
          pltpu.SemaphoreType.DMA,
      ],
      compiler_params=pltpu.CompilerParams(needs_layout_passes=False),
  )
  return f(keys, scores, x2d)


def _gmm_body(be_ref, vl_ref, x_ref, w_ref, b_ref, s_ref, o_ref):
  @pl.when(vl_ref[pl.program_id(0)] == 1)
  def _():
    xb = x_ref[...]
    wb = w_ref[...][0]
    acc = lax.dot_general(xb, wb, (((1,), (1,)), ((), ())),
                          preferred_element_type=jnp.float32)
    o_ref[...] = (acc + b_ref[...][0]) * s_ref[...][0, 0][:, None]


def _gmm(blk_e, vld, xs, expert_W, expert_b, ss):
  grid_spec = pltpu.PrefetchScalarGridSpec(
      num_scalar_prefetch=2,
      grid=(NBLK,),
      in_specs=[
          pl.BlockSpec((M_, D_IN), lambda i, be, vl: (i, 0)),
          pl.BlockSpec((1, D_OUT, D_IN), lambda i, be, vl: (be[i], 0, 0)),
          pl.BlockSpec((1, 1, D_OUT), lambda i, be, vl: (be[i], 0, 0)),
          pl.BlockSpec((1, 1, M_), lambda i, be, vl: (i, 0, 0)),
      ],
      out_specs=pl.BlockSpec((M_, D_OUT), lambda i, be, vl: (i, 0)),
  )
  return pl.pallas_call(
      _gmm_body,
      grid_spec=grid_spec,
      out_shape=jax.ShapeDtypeStruct((NPAD, D_OUT), jnp.float32),
  )(blk_e, vld, xs, expert_W, expert_b.reshape(E_, 1, D_OUT),
    ss.reshape(NBLK, 1, M_))


def _combine_body(y_hbm, pos_hbm, out_hbm,
                  i0_v, i1_v, r0_v, r1_v, ob_v, sem0, sem1):
  cid = lax.axis_index("c")
  sid = lax.axis_index("s")
  tb = (cid * NS + sid) * TPW

  def step(c, _):
    t0 = tb + c * L
    pltpu.sync_copy(pos_hbm.at[pl.ds(t0, L)], i0_v)
    pltpu.sync_copy(pos_hbm.at[pl.ds(T_ + t0, L)], i1_v)
    cp0 = pltpu.async_copy(y_hbm.at[i0_v], r0_v, sem0)
    cp1 = pltpu.async_copy(y_hbm.at[i1_v], r1_v, sem1)
    cp0.wait()
    cp1.wait()

    def add_col(cc, _):
      for rr in range(L):
        ob_v[rr, pl.ds(cc * L, L)] = (r0_v[rr, pl.ds(cc * L, L)] +
                                      r1_v[rr, pl.ds(cc * L, L)])
      return 0
    lax.fori_loop(0, D_OUT // L, add_col, 0)
    pltpu.sync_copy(ob_v, out_hbm.at[pl.ds(t0, L)])
    return 0
  lax.fori_loop(0, TPW // L, step, 0)


def _combine(y, pos):
  mesh = plsc.VectorSubcoreMesh(core_axis_name="c", subcore_axis_name="s",
                                num_cores=NC, num_subcores=NS)
  f = pl.kernel(
      _combine_body,
      out_type=jax.ShapeDtypeStruct((T_, D_OUT), jnp.float32),
      mesh=mesh,
      scratch_types=[
          pltpu.VMEM((L,), jnp.int32),
          pltpu.VMEM((L,), jnp.int32),
          pltpu.VMEM((L, D_OUT), jnp.float32),
          pltpu.VMEM((L, D_OUT), jnp.float32),
          pltpu.VMEM((L, D_OUT), jnp.float32),
          pltpu.SemaphoreType.DMA,
          pltpu.SemaphoreType.DMA,
      ],
      compiler_params=pltpu.CompilerParams(needs_layout_passes=False),
  )
  return f(y, pos)


@jax.jit
def kernel(x, expert_W, expert_b, gate_W, gate_b):
  x2d = x.reshape(T_, D_IN)
  scores, idx = _gating(x2d, gate_W, gate_b.reshape(1, E_))
  keys = idx.reshape(A_)
  scores_flat = scores.reshape(A_)
  xs, ss, pos, blk, vld = _route_gather(keys, scores_flat, x2d)
  y = _gmm(blk[:NBLK], vld[:NBLK], xs, expert_W, expert_b, ss)
  out = _combine(y, pos)
  return out.reshape(B_, S_, D_OUT)
